# baseline (speedup 1.0000x reference)
import functools

import jax
import jax.numpy as jnp
from jax import lax
from jax.experimental import pallas as pl
from jax.experimental.pallas import tpu as pltpu
from jax.experimental.pallas import tpu_sc as plsc

B = 16384
D = 64
NC = 2
NS = 16
KPT = B // NS
JPC = D // NC
ROWS = 1000000
MAIN = 999936
PIECE = MAIN // 2
NTAIL = ROWS - MAIN
SLC = 31232
SLC_LAST = PIECE - 15 * SLC
LANES = 16
GCH = 128
NG = KPT // GCH
NSL = KPT // LANES

_mesh = plsc.VectorSubcoreMesh(core_axis_name="c", subcore_axis_name="s")


@functools.partial(
    pl.kernel,
    mesh=_mesh,
    out_type=jax.ShapeDtypeStruct((D, B), jnp.float32),
    scratch_types=[
        pltpu.VMEM((KPT,), jnp.int32),
        pltpu.VMEM((KPT,), jnp.int32),
        pltpu.VMEM((KPT,), jnp.int32),
        pltpu.VMEM((2 * KPT,), jnp.float32),
        pltpu.VMEM((JPC, KPT), jnp.float32),
        pltpu.VMEM((NTAIL * D,), jnp.float32),
        pltpu.VMEM_SHARED((PIECE,), jnp.float32),
        pltpu.VMEM_SHARED((PIECE,), jnp.float32),
        pltpu.SemaphoreType.DMA,
        pltpu.SemaphoreType.DMA,
        pltpu.SemaphoreType.DMA,
    ],
    compiler_params=pltpu.CompilerParams(needs_layout_passes=False),
)
def _gather_tanh(idx_hbm, table_hbm, tail_hbm, out_hbm, idx_v, rel0_v,
                 rel1_v, tmp_v, acc_v, tail_v, ring_a, ring_b, sem_a,
                 sem_b, gsem):
    cid = lax.axis_index("c")
    sid = lax.axis_index("s")

    pltpu.sync_copy(idx_hbm.at[pl.ds(sid * KPT, KPT)], idx_v)
    pltpu.sync_copy(tail_hbm, tail_v)

    def fill(m, ring, sem, lo):
        o = 4 * cid + (m >> 3)
        b = m & 7
        src = table_hbm.at[o, b]

        @pl.when(sid < NS - 1)
        def _():
            pltpu.async_copy(
                src.at[pl.ds(lo + sid * SLC, SLC)],
                ring.at[pl.ds(sid * SLC, SLC)],
                sem,
            )

        @pl.when(sid == NS - 1)
        def _():
            pltpu.async_copy(
                src.at[pl.ds(lo + 15 * SLC, SLC_LAST)],
                ring.at[pl.ds(15 * SLC, SLC_LAST)],
                sem,
            )

    def wait_fill(ring, sem):
        @pl.when(sid < NS - 1)
        def _():
            pltpu.make_async_copy(
                table_hbm.at[0, 0, pl.ds(0, SLC)],
                ring.at[pl.ds(0, SLC)],
                sem,
            ).wait()

        @pl.when(sid == NS - 1)
        def _():
            pltpu.make_async_copy(
                table_hbm.at[0, 0, pl.ds(0, SLC_LAST)],
                ring.at[pl.ds(0, SLC_LAST)],
                sem,
            ).wait()

    mx = idx_v[pl.ds(0, LANES)]
    for s in range(NSL):
        sl = pl.ds(s * LANES, LANES)
        t = idx_v[sl]
        if s:
            mx = jnp.maximum(mx, t)
        rel0_v[sl] = jnp.minimum(t, PIECE - 1)
        rel1_v[sl] = jnp.minimum(jnp.maximum(t - PIECE, 0), PIECE - 1)
    has_tail = jnp.max(mx) >= MAIN

    fill(0, ring_a, sem_a, 0)
    fill(0, ring_b, sem_b, PIECE)

    def stage(m, _):
        wait_fill(ring_a, sem_a)
        wait_fill(ring_b, sem_b)
        plsc.subcore_barrier()

        for g in range(NG):
            pltpu.async_copy(
                ring_a.at[rel0_v.at[pl.ds(g * GCH, GCH)]],
                tmp_v.at[pl.ds(g * GCH, GCH)],
                gsem,
            )
        for g in range(NG):
            pltpu.async_copy(
                ring_b.at[rel1_v.at[pl.ds(g * GCH, GCH)]],
                tmp_v.at[pl.ds(KPT + g * GCH, GCH)],
                gsem,
            )
        pltpu.make_async_copy(
            table_hbm.at[0, 0, pl.ds(0, 2 * KPT)], tmp_v, gsem
        ).wait()

        for s in range(0):
            sl = pl.ds(s * LANES, LANES)
            x0 = tmp_v[sl]
            x1 = tmp_v[pl.ds(KPT + s * LANES, LANES)]
            x = jnp.where(idx_v[sl] >= PIECE, x1, x0)
            e = jnp.exp(x * 2.0)
            acc_v[m, sl] = 1.0 - 2.0 / (e + 1.0)

        plsc.subcore_barrier()

        @pl.when(m + 1 < JPC)
        def _():
            fill(m + 1, ring_a, sem_a, 0)
            fill(m + 1, ring_b, sem_b, PIECE)

        return 0

    lax.fori_loop(0, JPC, stage, 0)

    @pl.when(has_tail)
    def _():
        def fixup(m, _):
            jrow = JPC * cid + m
            for s in range(NSL):
                sl = pl.ds(s * LANES, LANES)
                t = idx_v[sl]
                toff = jrow * NTAIL + (t - MAIN)
                toff = jnp.minimum(jnp.maximum(toff, 0), NTAIL * D - 1)
                tv = plsc.load_gather(tail_v, [toff])
                e = jnp.exp(tv * 2.0)
                y = 1.0 - 2.0 / (e + 1.0)
                acc_v[m, sl] = jnp.where(t >= MAIN, y, acc_v[m, sl])
            return 0

        lax.fori_loop(0, JPC, fixup, 0)

    pltpu.sync_copy(
        acc_v,
        out_hbm.at[pl.ds(JPC * cid, JPC), pl.ds(sid * KPT, KPT)],
    )


def kernel(topics, table):
    tail = table[MAIN:].T.reshape(-1)
    out_t = _gather_tanh(
        topics.astype(jnp.int32), table.T.reshape(8, 8, ROWS), tail
    )
    return out_t.T.reshape(B, 1, D)


# speedup vs baseline: 2.4858x; 2.4858x over previous
import functools

import jax
import jax.numpy as jnp
from jax import lax
from jax.experimental import pallas as pl
from jax.experimental.pallas import tpu as pltpu
from jax.experimental.pallas import tpu_sc as plsc

B = 16384
D = 64
NC = 2
NS = 16
KPT = B // NS
JPC = D // NC
ROWS = 1000000
MAIN = 999936
NP = 3
PIECE = MAIN // NP
NB = 4
NST = JPC * NP
NTAIL = ROWS - MAIN
SLC = 20736
SLC_LAST = PIECE - 15 * SLC
LANES = 16
GCH = 128
NG = KPT // GCH
NSL = KPT // LANES

_mesh = plsc.VectorSubcoreMesh(core_axis_name="c", subcore_axis_name="s")


@functools.partial(
    pl.kernel,
    mesh=_mesh,
    out_type=jax.ShapeDtypeStruct((D, B), jnp.float32),
    scratch_types=[
        pltpu.VMEM((KPT,), jnp.int32),
        pltpu.VMEM((KPT,), jnp.int32),
        pltpu.VMEM((KPT,), jnp.int32),
        pltpu.VMEM((KPT,), jnp.int32),
        pltpu.VMEM((KPT,), jnp.float32),
        pltpu.VMEM((JPC, KPT), jnp.float32),
        pltpu.VMEM((NTAIL * D,), jnp.float32),
        pltpu.VMEM_SHARED((NB * PIECE,), jnp.float32),
        pltpu.SemaphoreType.DMA,
        pltpu.SemaphoreType.DMA,
        pltpu.SemaphoreType.DMA,
        pltpu.SemaphoreType.DMA,
        pltpu.SemaphoreType.DMA,
    ],
    compiler_params=pltpu.CompilerParams(needs_layout_passes=False),
)
def _gather_tanh(idx_hbm, table_hbm, tail_hbm, out_hbm, idx_v, piece_v,
                 rel_v, gidx_v, tmp_v, acc_v, tail_v, ring_sh, sem0,
                 sem1, sem2, sem3, gsem):
    cid = lax.axis_index("c")
    sid = lax.axis_index("s")
    sems = (sem0, sem1, sem2, sem3)

    pltpu.sync_copy(idx_hbm.at[pl.ds(sid * KPT, KPT)], idx_v)
    pltpu.sync_copy(tail_hbm, tail_v)

    def fill(m, p, seg):
        o = 4 * cid + (m >> 3)
        b = m & 7
        src = table_hbm.at[o, b]
        src_lo = p * PIECE
        dst_lo = seg * PIECE

        for sg in range(NB):
            @pl.when(seg == sg)
            def _(sg=sg):
                @pl.when(sid < NS - 1)
                def _():
                    pltpu.async_copy(
                        src.at[pl.ds(src_lo + sid * SLC, SLC)],
                        ring_sh.at[pl.ds(dst_lo + sid * SLC, SLC)],
                        sems[sg],
                    )

                @pl.when(sid == NS - 1)
                def _():
                    pltpu.async_copy(
                        src.at[pl.ds(src_lo + 15 * SLC, SLC_LAST)],
                        ring_sh.at[pl.ds(dst_lo + 15 * SLC, SLC_LAST)],
                        sems[sg],
                    )

    def wait_fill(seg):
        for sg in range(NB):
            @pl.when(seg == sg)
            def _(sg=sg):
                @pl.when(sid < NS - 1)
                def _():
                    pltpu.make_async_copy(
                        table_hbm.at[0, 0, pl.ds(0, SLC)],
                        ring_sh.at[pl.ds(0, SLC)],
                        sems[sg],
                    ).wait()

                @pl.when(sid == NS - 1)
                def _():
                    pltpu.make_async_copy(
                        table_hbm.at[0, 0, pl.ds(0, SLC_LAST)],
                        ring_sh.at[pl.ds(0, SLC_LAST)],
                        sems[sg],
                    ).wait()

    mx = idx_v[pl.ds(0, LANES)]
    for s in range(NSL):
        sl = pl.ds(s * LANES, LANES)
        t = idx_v[sl]
        if s:
            mx = jnp.maximum(mx, t)
        pv = (t >= PIECE).astype(jnp.int32) + (t >= 2 * PIECE).astype(
            jnp.int32
        )
        piece_v[sl] = pv
        rel_v[sl] = jnp.minimum(t - pv * PIECE, PIECE - 1)
    has_tail = jnp.max(mx) >= MAIN

    fill(0, 0, 0)
    fill(0, 1, 1)

    def stage(st, carry):
        m, p = carry
        seg = (3 * m + p) & (NB - 1)

        wait_fill(seg)

        @pl.when(p == NP - 1)
        def _():
            plsc.subcore_barrier()
            base = 3 * m
            for s in range(NSL):
                sl = pl.ds(s * LANES, LANES)
                sv = (base + piece_v[sl]) & (NB - 1)
                gidx_v[sl] = sv * PIECE + rel_v[sl]
            for g in range(NG):
                pltpu.async_copy(
                    ring_sh.at[gidx_v.at[pl.ds(g * GCH, GCH)]],
                    tmp_v.at[pl.ds(g * GCH, GCH)],
                    gsem,
                )
            pltpu.make_async_copy(
                table_hbm.at[0, 0, pl.ds(0, KPT)], tmp_v, gsem
            ).wait()
            plsc.subcore_barrier()

        m2 = jnp.where(p == NP - 1, m + 1, m)
        p2 = jnp.where(p == NP - 1, 0, p + 1)
        m3 = jnp.where(p2 == NP - 1, m2 + 1, m2)
        p3 = jnp.where(p2 == NP - 1, 0, p2 + 1)

        @pl.when(st + 2 < NST)
        def _():
            fill(m3, p3, (3 * m3 + p3) & (NB - 1))

        @pl.when(p == NP - 1)
        def _():
            for s in range(NSL):
                sl = pl.ds(s * LANES, LANES)
                x = tmp_v[sl]
                e = jnp.exp(x * 2.0)
                acc_v[m, sl] = 1.0 - 2.0 / (e + 1.0)

        return (m2, p2)

    lax.fori_loop(0, NST, stage, (0, 0))

    @pl.when(has_tail)
    def _():
        def fixup(m, _):
            jrow = JPC * cid + m
            for s in range(NSL):
                sl = pl.ds(s * LANES, LANES)
                t = idx_v[sl]
                toff = jrow * NTAIL + (t - MAIN)
                toff = jnp.minimum(jnp.maximum(toff, 0), NTAIL * D - 1)
                tv = plsc.load_gather(tail_v, [toff])
                e = jnp.exp(tv * 2.0)
                y = 1.0 - 2.0 / (e + 1.0)
                acc_v[m, sl] = jnp.where(t >= MAIN, y, acc_v[m, sl])
            return 0

        lax.fori_loop(0, JPC, fixup, 0)

    pltpu.sync_copy(
        acc_v,
        out_hbm.at[pl.ds(JPC * cid, JPC), pl.ds(sid * KPT, KPT)],
    )


def kernel(topics, table):
    tail = table[MAIN:].T.reshape(-1)
    out_t = _gather_tanh(
        topics.astype(jnp.int32), table.T.reshape(8, 8, ROWS), tail
    )
    return out_t.T.reshape(B, 1, D)
